# baseline (speedup 1.0000x reference)
import functools

import jax
import jax.numpy as jnp
from jax import lax
from jax.experimental import pallas as pl
from jax.experimental.pallas import tpu as pltpu
from jax.experimental.pallas import tpu_sc as plsc

B = 16384
D_ARTIST, D_ALBUM, D_GENRE = 64, 32, 16
H1, H2 = 256, 128

NC, NS = 2, 16
NW = NC * NS
BPW = B // NW
IDC = 128
NCHUNK = BPW // IDC

_sc_mesh = plsc.VectorSubcoreMesh(core_axis_name="c", subcore_axis_name="s")


@functools.partial(
    pl.kernel,
    out_type=[
        jax.ShapeDtypeStruct((B, D_ARTIST), jnp.float32),
        jax.ShapeDtypeStruct((B, D_ALBUM), jnp.float32),
        jax.ShapeDtypeStruct((B, D_GENRE), jnp.float32),
    ],
    mesh=_sc_mesh,
    scratch_types=[
        pltpu.VMEM((IDC,), jnp.int32),
        pltpu.VMEM((IDC,), jnp.int32),
        pltpu.VMEM((IDC,), jnp.int32),
        pltpu.SemaphoreType.DMA,
        pltpu.SemaphoreType.DMA,
        pltpu.SemaphoreType.DMA,
    ],
)
def _sc_gather(aid_h, bid_h, gid_h, ea_h, eb_h, eg_h,
               oa_h, ob_h, og_h,
               sa_v, sb_v, sg_v, sema, semb, semg):
    wid = lax.axis_index("s") * NC + lax.axis_index("c")
    base = wid * BPW

    for k in range(NCHUNK):
        off = base + k * IDC
        pltpu.sync_copy(aid_h.at[pl.ds(off, IDC)], sa_v)
        pltpu.sync_copy(bid_h.at[pl.ds(off, IDC)], sb_v)
        pltpu.sync_copy(gid_h.at[pl.ds(off, IDC)], sg_v)

        @pl.loop(0, IDC, step=16)
        def _(i):
            va = sa_v[pl.ds(i, 16)]
            vb = sb_v[pl.ds(i, 16)]
            vg = sg_v[pl.ds(i, 16)]
            for j in range(16):
                row = off + i + j
                pltpu.async_copy(ea_h.at[pl.ds(va[j], 1)],
                                 oa_h.at[pl.ds(row, 1)], sema)
                pltpu.async_copy(eb_h.at[pl.ds(vb[j], 1)],
                                 ob_h.at[pl.ds(row, 1)], semb)
                pltpu.async_copy(eg_h.at[pl.ds(vg[j], 1)],
                                 og_h.at[pl.ds(row, 1)], semg)

    pltpu.make_async_copy(ea_h.at[pl.ds(0, BPW)],
                          oa_h.at[pl.ds(base, BPW)], sema).wait()
    pltpu.make_async_copy(eb_h.at[pl.ds(0, BPW)],
                          ob_h.at[pl.ds(base, BPW)], semb).wait()
    pltpu.make_async_copy(eg_h.at[pl.ds(0, BPW)],
                          og_h.at[pl.ds(base, BPW)], semg).wait()


BLK = 2048


def _mlp_body(num_ref, ea_ref, eb_ref, eg_ref,
              w1n_ref, w1a_ref, w1b_ref, w1g_ref, b1_ref, w2_ref, b2_ref,
              o_ref):
    h = jnp.dot(num_ref[...], w1n_ref[...], preferred_element_type=jnp.float32)
    h += jnp.dot(ea_ref[...], w1a_ref[...], preferred_element_type=jnp.float32)
    h += jnp.dot(eb_ref[...], w1b_ref[...], preferred_element_type=jnp.float32)
    h += jnp.dot(eg_ref[...], w1g_ref[...], preferred_element_type=jnp.float32)
    h = jnp.maximum(h + b1_ref[...], 0.0)
    o = jnp.dot(h, w2_ref[...], preferred_element_type=jnp.float32)
    o_ref[...] = jnp.maximum(o + b2_ref[...], 0.0)


def _mlp(num, ea, eb, eg, w1n, w1a, w1b, w1g, b1, w2, b2):
    grid = (B // BLK,)
    full = lambda shape: pl.BlockSpec(shape, lambda i: (0, 0))
    return pl.pallas_call(
        _mlp_body,
        grid=grid,
        in_specs=[
            pl.BlockSpec((BLK, 9), lambda i: (i, 0)),
            pl.BlockSpec((BLK, D_ARTIST), lambda i: (i, 0)),
            pl.BlockSpec((BLK, D_ALBUM), lambda i: (i, 0)),
            pl.BlockSpec((BLK, D_GENRE), lambda i: (i, 0)),
            full((9, H1)),
            full((D_ARTIST, H1)),
            full((D_ALBUM, H1)),
            full((D_GENRE, H1)),
            full((1, H1)),
            full((H1, H2)),
            full((1, H2)),
        ],
        out_specs=pl.BlockSpec((BLK, H2), lambda i: (i, 0)),
        out_shape=jax.ShapeDtypeStruct((B, H2), jnp.float32),
    )(num, ea, eb, eg, w1n, w1a, w1b, w1g, b1, w2, b2)


def kernel(danceability, energy, loudness, speechiness, acousticness,
           instrumentalness, liveness, valence, tempo,
           artist_id, album_id, genre_id,
           E_artist, E_album, E_genre, W1, b1, W2, b2):
    ea, eb, eg = _sc_gather(artist_id, album_id, genre_id,
                            E_artist, E_album, E_genre)
    num = jnp.stack([danceability, energy, loudness, speechiness, acousticness,
                     instrumentalness, liveness, valence, tempo], axis=1)
    return _mlp(num, ea, eb, eg,
                W1[:9], W1[9:9 + D_ARTIST],
                W1[9 + D_ARTIST:9 + D_ARTIST + D_ALBUM],
                W1[9 + D_ARTIST + D_ALBUM:],
                b1.reshape(1, H1), W2, b2.reshape(1, H2))


# speedup vs baseline: 1.3079x; 1.3079x over previous
import functools

import jax
import jax.numpy as jnp
from jax import lax
from jax.experimental import pallas as pl
from jax.experimental.pallas import tpu as pltpu
from jax.experimental.pallas import tpu_sc as plsc

B = 16384
D_ARTIST, D_ALBUM, D_GENRE = 64, 32, 16
H1, H2 = 256, 128

NC, NS = 2, 16
NW = NC * NS
BPW = B // NW
WIN = 128
NWINDOW = BPW // WIN

_sc_mesh = plsc.VectorSubcoreMesh(core_axis_name="c", subcore_axis_name="s")


@functools.partial(
    pl.kernel,
    out_type=[
        jax.ShapeDtypeStruct((B, 128), jnp.float32),
        jax.ShapeDtypeStruct((B, 128), jnp.float32),
        jax.ShapeDtypeStruct((B, 128), jnp.float32),
    ],
    mesh=_sc_mesh,
    scratch_types=[
        pltpu.VMEM((1, WIN), jnp.int32),
        pltpu.VMEM((1, WIN), jnp.int32),
        pltpu.VMEM((1, WIN), jnp.int32),
        pltpu.VMEM((WIN, 128), jnp.float32),
        pltpu.VMEM((WIN, 128), jnp.float32),
        pltpu.VMEM((WIN, 128), jnp.float32),
        pltpu.SemaphoreType.DMA,
        pltpu.SemaphoreType.DMA,
        pltpu.SemaphoreType.DMA,
    ],
)
def _sc_gather(aidx_h, bidx_h, gidx_h, ap_h, bp_h, gp_h,
               oa_h, ob_h, og_h,
               ixa, ixb, ixg, ra, rb, rg, sema, semb, semg):
    wid = lax.axis_index("s") * NC + lax.axis_index("c")

    for j in range(NWINDOW):
        r = wid * NWINDOW + j
        pltpu.sync_copy(aidx_h.at[pl.ds(r, 1)], ixa)
        pltpu.sync_copy(bidx_h.at[pl.ds(r, 1)], ixb)
        pltpu.sync_copy(gidx_h.at[pl.ds(r, 1)], ixg)
        ca = pltpu.async_copy(ap_h.at[ixa.at[0]], ra, sema)
        cb = pltpu.async_copy(bp_h.at[ixb.at[0]], rb, semb)
        cg = pltpu.async_copy(gp_h.at[ixg.at[0]], rg, semg)
        ca.wait()
        cb.wait()
        cg.wait()
        base = wid * BPW + j * WIN
        pltpu.sync_copy(ra, oa_h.at[pl.ds(base, WIN)])
        pltpu.sync_copy(rb, ob_h.at[pl.ds(base, WIN)])
        pltpu.sync_copy(rg, og_h.at[pl.ds(base, WIN)])


BLK = 2048


def _subselect(packed, ids, d):
    out = jnp.zeros((BLK, d), jnp.float32)
    ngrp = 128 // d
    sel = ids & (ngrp - 1)
    for grp in range(ngrp):
        m = (sel == grp).astype(jnp.float32)
        out += m * packed[:, grp * d:(grp + 1) * d]
    return out


def _mlp_body(num_ref, ea_ref, eb_ref, eg_ref, aid_ref, bid_ref, gid_ref,
              w1n_ref, w1a_ref, w1b_ref, w1g_ref, b1_ref, w2_ref, b2_ref,
              o_ref):
    ea = _subselect(ea_ref[...], aid_ref[...], D_ARTIST)
    eb = _subselect(eb_ref[...], bid_ref[...], D_ALBUM)
    eg = _subselect(eg_ref[...], gid_ref[...], D_GENRE)
    h = jnp.dot(num_ref[...], w1n_ref[...], preferred_element_type=jnp.float32)
    h += jnp.dot(ea, w1a_ref[...], preferred_element_type=jnp.float32)
    h += jnp.dot(eb, w1b_ref[...], preferred_element_type=jnp.float32)
    h += jnp.dot(eg, w1g_ref[...], preferred_element_type=jnp.float32)
    h = jnp.maximum(h + b1_ref[...], 0.0)
    o = jnp.dot(h, w2_ref[...], preferred_element_type=jnp.float32)
    o_ref[...] = jnp.maximum(o + b2_ref[...], 0.0)


def _mlp(num, ea, eb, eg, aid, bid, gid, w1n, w1a, w1b, w1g, b1, w2, b2):
    grid = (B // BLK,)
    full = lambda shape: pl.BlockSpec(shape, lambda i: (0, 0))
    blk2 = lambda shape: pl.BlockSpec(shape, lambda i: (i, 0))
    return pl.pallas_call(
        _mlp_body,
        grid=grid,
        in_specs=[
            blk2((BLK, 9)),
            blk2((BLK, 128)),
            blk2((BLK, 128)),
            blk2((BLK, 128)),
            blk2((BLK, 1)),
            blk2((BLK, 1)),
            blk2((BLK, 1)),
            full((9, H1)),
            full((D_ARTIST, H1)),
            full((D_ALBUM, H1)),
            full((D_GENRE, H1)),
            full((1, H1)),
            full((H1, H2)),
            full((1, H2)),
        ],
        out_specs=blk2((BLK, H2)),
        out_shape=jax.ShapeDtypeStruct((B, H2), jnp.float32),
    )(num, ea, eb, eg, aid, bid, gid, w1n, w1a, w1b, w1g, b1, w2, b2)


def kernel(danceability, energy, loudness, speechiness, acousticness,
           instrumentalness, liveness, valence, tempo,
           artist_id, album_id, genre_id,
           E_artist, E_album, E_genre, W1, b1, W2, b2):
    ap = E_artist.reshape(500000, 128)
    bp = E_album.reshape(250000, 128)
    gp = E_genre.reshape(125, 128)
    aidx = (artist_id >> 1).reshape(NW * NWINDOW, WIN)
    bidx = (album_id >> 2).reshape(NW * NWINDOW, WIN)
    gidx = (genre_id >> 3).reshape(NW * NWINDOW, WIN)
    ea, eb, eg = _sc_gather(aidx, bidx, gidx, ap, bp, gp)
    num = jnp.stack([danceability, energy, loudness, speechiness, acousticness,
                     instrumentalness, liveness, valence, tempo], axis=1)
    return _mlp(num, ea, eb, eg,
                artist_id.reshape(B, 1), album_id.reshape(B, 1),
                genre_id.reshape(B, 1),
                W1[:9], W1[9:9 + D_ARTIST],
                W1[9 + D_ARTIST:9 + D_ARTIST + D_ALBUM],
                W1[9 + D_ARTIST + D_ALBUM:],
                b1.reshape(1, H1), W2, b2.reshape(1, H2))
